# baseline (speedup 1.0000x reference)
import functools

import jax
import jax.numpy as jnp
from jax import lax
from jax.experimental import pallas as pl
from jax.experimental.pallas import tpu as pltpu
from jax.experimental.pallas import tpu_sc as plsc

N = 10000
E = 160000
D = 256
R = 16
B = 256
H = 128
EPS = 1e-5

NPHASE = 2
DH = D // NPHASE
SPROWS = N + 8
STRIPE = 624
NTILES = 32
EDGES_PER_TILE = E // NTILES
CHUNK = 128
NCHUNKS = 40
PAD = NCHUNKS * CHUNK - EDGES_PER_TILE


def _idx_body(src_ref, et_ref, o_ref):
    o_ref[...] = (src_ref[...] * R + et_ref[...]) * NPHASE


def _edge_tables(src, etype, dst):
    gidx = pl.pallas_call(
        _idx_body,
        out_shape=jax.ShapeDtypeStruct((E // 128, 128), jnp.int32),
    )(src.reshape(E // 128, 128), etype.reshape(E // 128, 128))
    gidx_t = jnp.pad(gidx.reshape(NTILES, EDGES_PER_TILE), ((0, 0), (0, PAD)))
    dump = jnp.broadcast_to(N + jnp.arange(PAD, dtype=jnp.int32) % 8,
                            (NTILES, PAD))
    dst_t = jnp.concatenate(
        [dst.reshape(NTILES, EDGES_PER_TILE), dump], axis=1)
    return (gidx_t.reshape(NTILES, NCHUNKS, CHUNK),
            (gidx_t + 1).reshape(NTILES, NCHUNKS, CHUNK),
            dst_t.reshape(NTILES, NCHUNKS, CHUNK))


def _mm_body(x_ref, w_ref, o_ref):
    o_ref[...] = jnp.dot(x_ref[...], w_ref[0],
                         preferred_element_type=jnp.float32)


def _rel_transform(x, w_rel):
    rows = 2000
    return pl.pallas_call(
        _mm_body,
        grid=(N // rows, R),
        in_specs=[
            pl.BlockSpec((rows, D), lambda i, j: (i, 0)),
            pl.BlockSpec((1, D, D), lambda i, j: (j, 0, 0)),
        ],
        out_specs=pl.BlockSpec((rows, D), lambda i, j: (i, j)),
        out_shape=jax.ShapeDtypeStruct((N, R * D), jnp.float32),
    )(x, w_rel)


def _sc_aggregate(xr_q, gidx0_t, gidx1_t, dst_t):
    mesh = plsc.VectorSubcoreMesh(core_axis_name="c", subcore_axis_name="s")

    @functools.partial(
        pl.kernel,
        mesh=mesh,
        out_type=jax.ShapeDtypeStruct((2 * NPHASE * N, DH), jnp.float32),
        scratch_types=[
            pltpu.VMEM((NCHUNKS, CHUNK), jnp.int32),
            pltpu.VMEM((NCHUNKS, CHUNK), jnp.int32),
            pltpu.VMEM((NCHUNKS, CHUNK), jnp.int32),
            pltpu.VMEM((2, CHUNK, DH), jnp.float32),
            pltpu.VMEM((16, DH), jnp.float32),
            pltpu.VMEM_SHARED((SPROWS, DH), jnp.float32),
            pltpu.SemaphoreType.DMA,
            pltpu.SemaphoreType.DMA,
        ],
    )
    def k(xr_hbm, gidx0_hbm, gidx1_hbm, dst_hbm, out_hbm,
          idx0, idx1, ldst2d, rows2, zero_v, agg_sh, sem0, sem1):
        cid = lax.axis_index("c")
        sid = lax.axis_index("s")
        tid = cid * 16 + sid

        pltpu.sync_copy(gidx0_hbm.at[tid], idx0)
        pltpu.sync_copy(gidx1_hbm.at[tid], idx1)
        pltpu.sync_copy(dst_hbm.at[tid], ldst2d)

        nz = DH // 16

        def zfill(i, c):
            zero_v[i // nz, pl.ds((i % nz) * 16, 16)] = jnp.zeros(
                (16,), jnp.float32)
            return c
        lax.fori_loop(0, 16 * nz, zfill, 0)

        base = sid * STRIPE

        def run_phase(q, idx2d):
            def zcopy(i, c):
                pltpu.sync_copy(zero_v, agg_sh.at[pl.ds(base + i * 16, 16)])
                return c
            lax.fori_loop(0, STRIPE // 16, zcopy, 0)

            @pl.when(sid == 15)
            def _():
                pltpu.sync_copy(zero_v, agg_sh.at[pl.ds(16 * STRIPE, 16)])
                pltpu.sync_copy(zero_v.at[pl.ds(0, 8)], agg_sh.at[pl.ds(N, 8)])
            plsc.subcore_barrier()

            pltpu.async_copy(xr_hbm.at[idx2d.at[0]], rows2.at[0], sem0)

            def chunk_body(ch, carry):
                nxt = ch + 1

                @pl.when((nxt < NCHUNKS) & (nxt % 2 == 0))
                def _():
                    pltpu.async_copy(xr_hbm.at[idx2d.at[nxt]], rows2.at[0],
                                     sem0)

                @pl.when((nxt < NCHUNKS) & (nxt % 2 == 1))
                def _():
                    pltpu.async_copy(xr_hbm.at[idx2d.at[nxt]], rows2.at[1],
                                     sem1)

                @pl.when(ch % 2 == 0)
                def _():
                    pltpu.make_async_copy(xr_hbm.at[pl.ds(0, CHUNK)],
                                          rows2.at[0], sem0).wait()
                    pltpu.sync_copy(rows2.at[0], agg_sh.at[ldst2d.at[ch]],
                                    add=True)

                @pl.when(ch % 2 == 1)
                def _():
                    pltpu.make_async_copy(xr_hbm.at[pl.ds(0, CHUNK)],
                                          rows2.at[1], sem1).wait()
                    pltpu.sync_copy(rows2.at[1], agg_sh.at[ldst2d.at[ch]],
                                    add=True)
                return carry
            lax.fori_loop(0, NCHUNKS, chunk_body, 0)
            plsc.subcore_barrier()

            obase = (cid * NPHASE + q) * N
            pltpu.sync_copy(agg_sh.at[pl.ds(base, STRIPE)],
                            out_hbm.at[pl.ds(obase + base, STRIPE)])

            @pl.when(sid == 15)
            def _():
                pltpu.sync_copy(
                    agg_sh.at[pl.ds(16 * STRIPE, N - 16 * STRIPE)],
                    out_hbm.at[pl.ds(obase + 16 * STRIPE, N - 16 * STRIPE)])
            plsc.subcore_barrier()

        run_phase(0, idx0)
        run_phase(1, idx1)

    return k(xr_q, gidx0_t, gidx1_t, dst_t)


def _post_body(p_ref, x_ref, gid_ref,
               b_rel, res_W, res_b, bn_g, bn_b,
               att_w_row, att_b,
               fc1_W, fc1_b, bn1_g, bn1_b,
               fc2_W, fc2_b, bn2_g, bn2_b,
               fc3_W, fc3_b, bn3_g, bn3_b,
               out_W, out_b, o_ref):
    x = x_ref[...]
    p = p_ref[...]
    agg = jnp.concatenate(
        [p[0, q] + p[1, q] for q in range(NPHASE)], axis=1)
    h = jnp.maximum(agg + b_rel[...], 0.0)
    res = jnp.maximum(
        jnp.dot(x, res_W[...], preferred_element_type=jnp.float32)
        + res_b[...], 0.0)
    h = h + res
    m = jnp.mean(h, axis=0, keepdims=True)
    v = jnp.mean((h - m) * (h - m), axis=0, keepdims=True)
    h = (h - m) / jnp.sqrt(v + EPS) * bn_g[...] + bn_b[...]
    z = jnp.sum(h * att_w_row[...], axis=1, keepdims=True) + att_b[...]
    w = 1.0 / (1.0 + jnp.exp(-z))
    hw = h * w
    sel = (lax.broadcasted_iota(jnp.int32, (B, N), 0)
           == gid_ref[...]).astype(jnp.float32)
    g = jnp.dot(sel, hw, preferred_element_type=jnp.float32)

    def fc(t, Wk, bk, gk, btk):
        y = jnp.maximum(
            jnp.dot(t, Wk[...], preferred_element_type=jnp.float32)
            + bk[...], 0.0)
        mm = jnp.mean(y, axis=0, keepdims=True)
        vv = jnp.mean((y - mm) * (y - mm), axis=0, keepdims=True)
        return (y - mm) / jnp.sqrt(vv + EPS) * gk[...] + btk[...]

    h1 = fc(g, fc1_W, fc1_b, bn1_g, bn1_b)
    h2 = fc(h1, fc2_W, fc2_b, bn2_g, bn2_b)
    h3 = fc(h2, fc3_W, fc3_b, bn3_g, bn3_b)
    o_ref[...] = (jnp.dot(h3, out_W[...], preferred_element_type=jnp.float32)
                  + out_b[...])


def _postprocess(partials, x, gid2d, p):
    args = (
        partials, x, gid2d,
        p['b_rel'].reshape(1, D), p['res_W'], p['res_b'].reshape(1, D),
        p['bn_g'].reshape(1, D), p['bn_b'].reshape(1, D),
        p['att_W'].reshape(1, D), p['att_b'].reshape(1, 1),
        p['fc1_W'], p['fc1_b'].reshape(1, H),
        p['bn1_g'].reshape(1, H), p['bn1_b'].reshape(1, H),
        p['fc2_W'], p['fc2_b'].reshape(1, H),
        p['bn2_g'].reshape(1, H), p['bn2_b'].reshape(1, H),
        p['fc3_W'], p['fc3_b'].reshape(1, H),
        p['bn3_g'].reshape(1, H), p['bn3_b'].reshape(1, H),
        p['out_W'], p['out_b'].reshape(1, 1),
    )
    return pl.pallas_call(
        _post_body,
        out_shape=jax.ShapeDtypeStruct((B, 1), jnp.float32),
    )(*args)


def kernel(node_feats, params, edge_index, etype, graph_ids):
    gidx0_t, gidx1_t, dst_t = _edge_tables(edge_index[0], etype,
                                           edge_index[1])
    xr = _rel_transform(node_feats, params['W_rel'])
    partials = _sc_aggregate(xr.reshape(N * R * NPHASE, DH),
                             gidx0_t, gidx1_t, dst_t)
    gid2d = graph_ids.reshape(1, N)
    return _postprocess(partials.reshape(2, NPHASE, N, DH),
                        node_feats, gid2d, params)


# speedup vs baseline: 1.1058x; 1.1058x over previous
import functools

import jax
import jax.numpy as jnp
from jax import lax
from jax.experimental import pallas as pl
from jax.experimental.pallas import tpu as pltpu
from jax.experimental.pallas import tpu_sc as plsc

N = 10000
E = 160000
D = 256
R = 16
B = 256
H = 128
EPS = 1e-5

NPHASE = 2
DH = D // NPHASE
SPROWS = N + 8
STRIPE = 624
NTILES = 32
EDGES_PER_TILE = E // NTILES
CHUNK = 96
NCHUNKS = 53
PAD = NCHUNKS * CHUNK - EDGES_PER_TILE


def _idx_body(src_ref, et_ref, o_ref):
    o_ref[...] = (src_ref[...] * R + et_ref[...]) * NPHASE


def _edge_tables(src, etype, dst):
    gidx = pl.pallas_call(
        _idx_body,
        out_shape=jax.ShapeDtypeStruct((E // 128, 128), jnp.int32),
    )(src.reshape(E // 128, 128), etype.reshape(E // 128, 128))
    gidx_t = jnp.pad(gidx.reshape(NTILES, EDGES_PER_TILE), ((0, 0), (0, PAD)))
    dump = jnp.broadcast_to(N + jnp.arange(PAD, dtype=jnp.int32) % 8,
                            (NTILES, PAD))
    dst_t = jnp.concatenate(
        [dst.reshape(NTILES, EDGES_PER_TILE), dump], axis=1)
    return (gidx_t.reshape(NTILES, NCHUNKS, CHUNK),
            (gidx_t + 1).reshape(NTILES, NCHUNKS, CHUNK),
            dst_t.reshape(NTILES, NCHUNKS, CHUNK))


def _mm_body(x_ref, w_ref, o_ref):
    o_ref[...] = jnp.dot(x_ref[...], w_ref[0],
                         preferred_element_type=jnp.float32)


def _rel_transform(x, w_rel):
    rows = 2000
    return pl.pallas_call(
        _mm_body,
        grid=(N // rows, R),
        in_specs=[
            pl.BlockSpec((rows, D), lambda i, j: (i, 0)),
            pl.BlockSpec((1, D, D), lambda i, j: (j, 0, 0)),
        ],
        out_specs=pl.BlockSpec((rows, D), lambda i, j: (i, j)),
        out_shape=jax.ShapeDtypeStruct((N, R * D), jnp.float32),
    )(x, w_rel)


def _sc_aggregate(xr_q, gidx0_t, gidx1_t, dst_t):
    mesh = plsc.VectorSubcoreMesh(core_axis_name="c", subcore_axis_name="s")

    @functools.partial(
        pl.kernel,
        mesh=mesh,
        out_type=jax.ShapeDtypeStruct((2 * NPHASE * N, DH), jnp.float32),
        scratch_types=[
            pltpu.VMEM((NCHUNKS, CHUNK), jnp.int32),
            pltpu.VMEM((NCHUNKS, CHUNK), jnp.int32),
            pltpu.VMEM((NCHUNKS, CHUNK), jnp.int32),
            pltpu.VMEM((2, CHUNK, DH), jnp.float32),
            pltpu.VMEM((16, DH), jnp.float32),
            pltpu.VMEM_SHARED((SPROWS, DH), jnp.float32),
            pltpu.SemaphoreType.DMA,
            pltpu.SemaphoreType.DMA,
        ],
    )
    def k(xr_hbm, gidx0_hbm, gidx1_hbm, dst_hbm, out_hbm,
          idx0, idx1, ldst2d, rows2, zero_v, agg_sh, sem0, sem1):
        cid = lax.axis_index("c")
        sid = lax.axis_index("s")
        tid = cid * 16 + sid

        pltpu.sync_copy(gidx0_hbm.at[tid], idx0)
        pltpu.sync_copy(gidx1_hbm.at[tid], idx1)
        pltpu.sync_copy(dst_hbm.at[tid], ldst2d)

        nz = DH // 16

        def zfill(i, c):
            zero_v[i // nz, pl.ds((i % nz) * 16, 16)] = jnp.zeros(
                (16,), jnp.float32)
            return c
        lax.fori_loop(0, 16 * nz, zfill, 0)

        base = sid * STRIPE

        def run_phase(q, idx2d):
            def zcopy(i, c):
                pltpu.sync_copy(zero_v, agg_sh.at[pl.ds(base + i * 16, 16)])
                return c
            lax.fori_loop(0, STRIPE // 16, zcopy, 0)

            @pl.when(sid == 15)
            def _():
                pltpu.sync_copy(zero_v, agg_sh.at[pl.ds(16 * STRIPE, 16)])
                pltpu.sync_copy(zero_v.at[pl.ds(0, 8)], agg_sh.at[pl.ds(N, 8)])
            plsc.subcore_barrier()

            pltpu.async_copy(xr_hbm.at[idx2d.at[0]], rows2.at[0], sem0)

            def chunk_body(ch, carry):
                nxt = ch + 1

                @pl.when((nxt < NCHUNKS) & (nxt % 2 == 0))
                def _():
                    pltpu.async_copy(xr_hbm.at[idx2d.at[nxt]], rows2.at[0],
                                     sem0)

                @pl.when((nxt < NCHUNKS) & (nxt % 2 == 1))
                def _():
                    pltpu.async_copy(xr_hbm.at[idx2d.at[nxt]], rows2.at[1],
                                     sem1)

                @pl.when(ch % 2 == 0)
                def _():
                    pltpu.make_async_copy(xr_hbm.at[pl.ds(0, CHUNK)],
                                          rows2.at[0], sem0).wait()
                    pltpu.sync_copy(rows2.at[0], agg_sh.at[ldst2d.at[ch]],
                                    add=True)

                @pl.when(ch % 2 == 1)
                def _():
                    pltpu.make_async_copy(xr_hbm.at[pl.ds(0, CHUNK)],
                                          rows2.at[1], sem1).wait()
                    pltpu.sync_copy(rows2.at[1], agg_sh.at[ldst2d.at[ch]],
                                    add=True)
                return carry
            lax.fori_loop(0, NCHUNKS, chunk_body, 0)
            plsc.subcore_barrier()

            obase = (cid * NPHASE + q) * N
            pltpu.sync_copy(agg_sh.at[pl.ds(base, STRIPE)],
                            out_hbm.at[pl.ds(obase + base, STRIPE)])

            @pl.when(sid == 15)
            def _():
                pltpu.sync_copy(
                    agg_sh.at[pl.ds(16 * STRIPE, N - 16 * STRIPE)],
                    out_hbm.at[pl.ds(obase + 16 * STRIPE, N - 16 * STRIPE)])
            plsc.subcore_barrier()

        run_phase(0, idx0)
        run_phase(1, idx1)

    return k(xr_q, gidx0_t, gidx1_t, dst_t)


def _post_body(p_ref, x_ref, gid_ref,
               b_rel, res_W, res_b, bn_g, bn_b,
               att_w_row, att_b,
               fc1_W, fc1_b, bn1_g, bn1_b,
               fc2_W, fc2_b, bn2_g, bn2_b,
               fc3_W, fc3_b, bn3_g, bn3_b,
               out_W, out_b, o_ref):
    x = x_ref[...]
    p = p_ref[...]
    agg = jnp.concatenate(
        [p[0, q] + p[1, q] for q in range(NPHASE)], axis=1)
    h = jnp.maximum(agg + b_rel[...], 0.0)
    res = jnp.maximum(
        jnp.dot(x, res_W[...], preferred_element_type=jnp.float32)
        + res_b[...], 0.0)
    h = h + res
    m = jnp.mean(h, axis=0, keepdims=True)
    v = jnp.mean((h - m) * (h - m), axis=0, keepdims=True)
    h = (h - m) / jnp.sqrt(v + EPS) * bn_g[...] + bn_b[...]
    z = jnp.sum(h * att_w_row[...], axis=1, keepdims=True) + att_b[...]
    w = 1.0 / (1.0 + jnp.exp(-z))
    hw = h * w
    sel = (lax.broadcasted_iota(jnp.int32, (B, N), 0)
           == gid_ref[...]).astype(jnp.float32)
    g = jnp.dot(sel, hw, preferred_element_type=jnp.float32)

    def fc(t, Wk, bk, gk, btk):
        y = jnp.maximum(
            jnp.dot(t, Wk[...], preferred_element_type=jnp.float32)
            + bk[...], 0.0)
        mm = jnp.mean(y, axis=0, keepdims=True)
        vv = jnp.mean((y - mm) * (y - mm), axis=0, keepdims=True)
        return (y - mm) / jnp.sqrt(vv + EPS) * gk[...] + btk[...]

    h1 = fc(g, fc1_W, fc1_b, bn1_g, bn1_b)
    h2 = fc(h1, fc2_W, fc2_b, bn2_g, bn2_b)
    h3 = fc(h2, fc3_W, fc3_b, bn3_g, bn3_b)
    o_ref[...] = (jnp.dot(h3, out_W[...], preferred_element_type=jnp.float32)
                  + out_b[...])


def _postprocess(partials, x, gid2d, p):
    args = (
        partials, x, gid2d,
        p['b_rel'].reshape(1, D), p['res_W'], p['res_b'].reshape(1, D),
        p['bn_g'].reshape(1, D), p['bn_b'].reshape(1, D),
        p['att_W'].reshape(1, D), p['att_b'].reshape(1, 1),
        p['fc1_W'], p['fc1_b'].reshape(1, H),
        p['bn1_g'].reshape(1, H), p['bn1_b'].reshape(1, H),
        p['fc2_W'], p['fc2_b'].reshape(1, H),
        p['bn2_g'].reshape(1, H), p['bn2_b'].reshape(1, H),
        p['fc3_W'], p['fc3_b'].reshape(1, H),
        p['bn3_g'].reshape(1, H), p['bn3_b'].reshape(1, H),
        p['out_W'], p['out_b'].reshape(1, 1),
    )
    return pl.pallas_call(
        _post_body,
        out_shape=jax.ShapeDtypeStruct((B, 1), jnp.float32),
    )(*args)


def kernel(node_feats, params, edge_index, etype, graph_ids):
    gidx0_t, gidx1_t, dst_t = _edge_tables(edge_index[0], etype,
                                           edge_index[1])
    xr = _rel_transform(node_feats, params['W_rel'])
    partials = _sc_aggregate(xr.reshape(N * R * NPHASE, DH),
                             gidx0_t, gidx1_t, dst_t)
    gid2d = graph_ids.reshape(1, N)
    return _postprocess(partials.reshape(2, NPHASE, N, DH),
                        node_feats, gid2d, params)


# speedup vs baseline: 1.5291x; 1.3828x over previous
import functools

import jax
import jax.numpy as jnp
from jax import lax
from jax.experimental import pallas as pl
from jax.experimental.pallas import tpu as pltpu
from jax.experimental.pallas import tpu_sc as plsc

N = 10000
E = 160000
D = 256
R = 16
B = 256
H = 128
EPS = 1e-5

NPHASE = 2
DH = D // NPHASE
SPROWS = N + 8
STRIPE = 624
NTILES = 32
EDGES_PER_TILE = E // NTILES
CHUNK = 96
NCHUNKS = 53
PAD = NCHUNKS * CHUNK - EDGES_PER_TILE


def _idx_body(src_ref, et_ref, o_ref):
    o_ref[...] = et_ref[...] * N + src_ref[...]


def _edge_tables(src, etype, dst):
    gidx = pl.pallas_call(
        _idx_body,
        out_shape=jax.ShapeDtypeStruct((E // 128, 128), jnp.int32),
    )(src.reshape(E // 128, 128), etype.reshape(E // 128, 128))
    gidx_t = jnp.pad(gidx.reshape(NTILES, EDGES_PER_TILE), ((0, 0), (0, PAD)))
    dump = jnp.broadcast_to(N + jnp.arange(PAD, dtype=jnp.int32) % 8,
                            (NTILES, PAD))
    dst_t = jnp.concatenate(
        [dst.reshape(NTILES, EDGES_PER_TILE), dump], axis=1)
    return (gidx_t.reshape(NTILES, NCHUNKS, CHUNK),
            dst_t.reshape(NTILES, NCHUNKS, CHUNK))


def _mm_body(x_ref, w_ref, o0_ref, o1_ref):
    r = jnp.dot(x_ref[...], w_ref[0], preferred_element_type=jnp.float32)
    o0_ref[...] = r[:, :DH]
    o1_ref[...] = r[:, DH:]


def _rel_transform(x, w_rel):
    rows = 2000
    nblk = N // rows
    return pl.pallas_call(
        _mm_body,
        grid=(nblk, R),
        in_specs=[
            pl.BlockSpec((rows, D), lambda i, j: (i, 0)),
            pl.BlockSpec((1, D, D), lambda i, j: (j, 0, 0)),
        ],
        out_specs=[
            pl.BlockSpec((rows, DH), lambda i, j: (j * nblk + i, 0)),
            pl.BlockSpec((rows, DH), lambda i, j: (j * nblk + i, 0)),
        ],
        out_shape=[
            jax.ShapeDtypeStruct((R * N, DH), jnp.float32),
            jax.ShapeDtypeStruct((R * N, DH), jnp.float32),
        ],
    )(x, w_rel)


def _sc_aggregate(xr0, xr1, gidx_t, dst_t):
    mesh = plsc.VectorSubcoreMesh(core_axis_name="c", subcore_axis_name="s")

    @functools.partial(
        pl.kernel,
        mesh=mesh,
        out_type=jax.ShapeDtypeStruct((2 * NPHASE * N, DH), jnp.float32),
        scratch_types=[
            pltpu.VMEM((NCHUNKS, CHUNK), jnp.int32),
            pltpu.VMEM((NCHUNKS, CHUNK), jnp.int32),
            pltpu.VMEM((2, CHUNK, DH), jnp.float32),
            pltpu.VMEM((16, DH), jnp.float32),
            pltpu.VMEM_SHARED((SPROWS, DH), jnp.float32),
            pltpu.SemaphoreType.DMA,
            pltpu.SemaphoreType.DMA,
        ],
    )
    def k(xr0_hbm, xr1_hbm, gidx_hbm, dst_hbm, out_hbm,
          idx2d, ldst2d, rows2, zero_v, agg_sh, sem0, sem1):
        cid = lax.axis_index("c")
        sid = lax.axis_index("s")
        tid = cid * 16 + sid

        pltpu.sync_copy(gidx_hbm.at[tid], idx2d)
        pltpu.sync_copy(dst_hbm.at[tid], ldst2d)

        nz = DH // 16

        def zfill(i, c):
            zero_v[i // nz, pl.ds((i % nz) * 16, 16)] = jnp.zeros(
                (16,), jnp.float32)
            return c
        lax.fori_loop(0, 16 * nz, zfill, 0)

        base = sid * STRIPE

        def run_phase(q, xr_hbm):
            def zcopy(i, c):
                pltpu.sync_copy(zero_v, agg_sh.at[pl.ds(base + i * 16, 16)])
                return c
            lax.fori_loop(0, STRIPE // 16, zcopy, 0)

            @pl.when(sid == 15)
            def _():
                pltpu.sync_copy(zero_v, agg_sh.at[pl.ds(16 * STRIPE, 16)])
                pltpu.sync_copy(zero_v.at[pl.ds(0, 8)], agg_sh.at[pl.ds(N, 8)])
            plsc.subcore_barrier()

            pltpu.async_copy(xr_hbm.at[idx2d.at[0]], rows2.at[0], sem0)

            def chunk_body(ch, carry):
                nxt = ch + 1

                @pl.when((nxt < NCHUNKS) & (nxt % 2 == 0))
                def _():
                    pltpu.async_copy(xr_hbm.at[idx2d.at[nxt]], rows2.at[0],
                                     sem0)

                @pl.when((nxt < NCHUNKS) & (nxt % 2 == 1))
                def _():
                    pltpu.async_copy(xr_hbm.at[idx2d.at[nxt]], rows2.at[1],
                                     sem1)

                @pl.when(ch % 2 == 0)
                def _():
                    pltpu.make_async_copy(xr_hbm.at[pl.ds(0, CHUNK)],
                                          rows2.at[0], sem0).wait()
                    pltpu.sync_copy(rows2.at[0], agg_sh.at[ldst2d.at[ch]],
                                    add=True)

                @pl.when(ch % 2 == 1)
                def _():
                    pltpu.make_async_copy(xr_hbm.at[pl.ds(0, CHUNK)],
                                          rows2.at[1], sem1).wait()
                    pltpu.sync_copy(rows2.at[1], agg_sh.at[ldst2d.at[ch]],
                                    add=True)
                return carry
            lax.fori_loop(0, NCHUNKS, chunk_body, 0)
            plsc.subcore_barrier()

            obase = (cid * NPHASE + q) * N
            pltpu.sync_copy(agg_sh.at[pl.ds(base, STRIPE)],
                            out_hbm.at[pl.ds(obase + base, STRIPE)])

            @pl.when(sid == 15)
            def _():
                pltpu.sync_copy(
                    agg_sh.at[pl.ds(16 * STRIPE, N - 16 * STRIPE)],
                    out_hbm.at[pl.ds(obase + 16 * STRIPE, N - 16 * STRIPE)])
            plsc.subcore_barrier()

        run_phase(0, xr0_hbm)
        run_phase(1, xr1_hbm)

    return k(xr0, xr1, gidx_t, dst_t)


def _post_body(p_ref, x_ref, gid_ref,
               b_rel, res_W, res_b, bn_g, bn_b,
               att_w_row, att_b,
               fc1_W, fc1_b, bn1_g, bn1_b,
               fc2_W, fc2_b, bn2_g, bn2_b,
               fc3_W, fc3_b, bn3_g, bn3_b,
               out_W, out_b, o_ref):
    x = x_ref[...]
    p = p_ref[...]
    agg = jnp.concatenate(
        [p[0, q] + p[1, q] for q in range(NPHASE)], axis=1)
    h = jnp.maximum(agg + b_rel[...], 0.0)
    res = jnp.maximum(
        jnp.dot(x, res_W[...], preferred_element_type=jnp.float32)
        + res_b[...], 0.0)
    h = h + res
    m = jnp.mean(h, axis=0, keepdims=True)
    v = jnp.mean((h - m) * (h - m), axis=0, keepdims=True)
    h = (h - m) / jnp.sqrt(v + EPS) * bn_g[...] + bn_b[...]
    z = jnp.sum(h * att_w_row[...], axis=1, keepdims=True) + att_b[...]
    w = 1.0 / (1.0 + jnp.exp(-z))
    hw = h * w
    sel = (lax.broadcasted_iota(jnp.int32, (B, N), 0)
           == gid_ref[...]).astype(jnp.float32)
    g = jnp.dot(sel, hw, preferred_element_type=jnp.float32)

    def fc(t, Wk, bk, gk, btk):
        y = jnp.maximum(
            jnp.dot(t, Wk[...], preferred_element_type=jnp.float32)
            + bk[...], 0.0)
        mm = jnp.mean(y, axis=0, keepdims=True)
        vv = jnp.mean((y - mm) * (y - mm), axis=0, keepdims=True)
        return (y - mm) / jnp.sqrt(vv + EPS) * gk[...] + btk[...]

    h1 = fc(g, fc1_W, fc1_b, bn1_g, bn1_b)
    h2 = fc(h1, fc2_W, fc2_b, bn2_g, bn2_b)
    h3 = fc(h2, fc3_W, fc3_b, bn3_g, bn3_b)
    o_ref[...] = (jnp.dot(h3, out_W[...], preferred_element_type=jnp.float32)
                  + out_b[...])


def _postprocess(partials, x, gid2d, p):
    args = (
        partials, x, gid2d,
        p['b_rel'].reshape(1, D), p['res_W'], p['res_b'].reshape(1, D),
        p['bn_g'].reshape(1, D), p['bn_b'].reshape(1, D),
        p['att_W'].reshape(1, D), p['att_b'].reshape(1, 1),
        p['fc1_W'], p['fc1_b'].reshape(1, H),
        p['bn1_g'].reshape(1, H), p['bn1_b'].reshape(1, H),
        p['fc2_W'], p['fc2_b'].reshape(1, H),
        p['bn2_g'].reshape(1, H), p['bn2_b'].reshape(1, H),
        p['fc3_W'], p['fc3_b'].reshape(1, H),
        p['bn3_g'].reshape(1, H), p['bn3_b'].reshape(1, H),
        p['out_W'], p['out_b'].reshape(1, 1),
    )
    return pl.pallas_call(
        _post_body,
        out_shape=jax.ShapeDtypeStruct((B, 1), jnp.float32),
    )(*args)


def kernel(node_feats, params, edge_index, etype, graph_ids):
    gidx_t, dst_t = _edge_tables(edge_index[0], etype, edge_index[1])
    xr0, xr1 = _rel_transform(node_feats, params['W_rel'])
    partials = _sc_aggregate(xr0, xr1, gidx_t, dst_t)
    gid2d = graph_ids.reshape(1, N)
    return _postprocess(partials.reshape(2, NPHASE, N, DH),
                        node_feats, gid2d, params)


# speedup vs baseline: 2.0954x; 1.3704x over previous
import functools

import jax
import jax.numpy as jnp
from jax import lax
from jax.experimental import pallas as pl
from jax.experimental.pallas import tpu as pltpu
from jax.experimental.pallas import tpu_sc as plsc

N = 10000
E = 160000
D = 256
R = 16
B = 256
H = 128
EPS = 1e-5

NPHASE = 2
DH = D // NPHASE
SPROWS = N + 8
STRIPE = 624
NTILES = 32
EDGES_PER_TILE = E // NTILES
CHUNK = 80
NCHUNKS = 63
PAD = NCHUNKS * CHUNK - EDGES_PER_TILE


def _idx_body(src_ref, et_ref, o_ref):
    o_ref[...] = et_ref[...] * N + src_ref[...]


def _edge_tables(src, etype, dst):
    gidx = pl.pallas_call(
        _idx_body,
        out_shape=jax.ShapeDtypeStruct((E // 128, 128), jnp.int32),
    )(src.reshape(E // 128, 128), etype.reshape(E // 128, 128))
    gidx_t = jnp.pad(gidx.reshape(NTILES, EDGES_PER_TILE), ((0, 0), (0, PAD)))
    dump = jnp.broadcast_to(N + jnp.arange(PAD, dtype=jnp.int32) % 8,
                            (NTILES, PAD))
    dst_t = jnp.concatenate(
        [dst.reshape(NTILES, EDGES_PER_TILE), dump], axis=1)
    return (gidx_t.reshape(NTILES, NCHUNKS, CHUNK),
            dst_t.reshape(NTILES, NCHUNKS, CHUNK))


def _mm_body(x_ref, w_ref, o0_ref, o1_ref):
    r = jnp.dot(x_ref[...], w_ref[0], preferred_element_type=jnp.float32)
    o0_ref[...] = r[:, :DH]
    o1_ref[...] = r[:, DH:]


def _rel_transform(x, w_rel):
    rows = 2000
    nblk = N // rows
    return pl.pallas_call(
        _mm_body,
        grid=(nblk, R),
        in_specs=[
            pl.BlockSpec((rows, D), lambda i, j: (i, 0)),
            pl.BlockSpec((1, D, D), lambda i, j: (j, 0, 0)),
        ],
        out_specs=[
            pl.BlockSpec((rows, DH), lambda i, j: (j * nblk + i, 0)),
            pl.BlockSpec((rows, DH), lambda i, j: (j * nblk + i, 0)),
        ],
        out_shape=[
            jax.ShapeDtypeStruct((R * N, DH), jnp.float32),
            jax.ShapeDtypeStruct((R * N, DH), jnp.float32),
        ],
    )(x, w_rel)


def _sc_aggregate(xr0, xr1, gidx_t, dst_t):
    mesh = plsc.VectorSubcoreMesh(core_axis_name="c", subcore_axis_name="s")

    @functools.partial(
        pl.kernel,
        mesh=mesh,
        out_type=jax.ShapeDtypeStruct((2 * NPHASE * N, DH), jnp.float32),
        scratch_types=[
            pltpu.VMEM((NCHUNKS, CHUNK), jnp.int32),
            pltpu.VMEM((NCHUNKS, CHUNK), jnp.int32),
            pltpu.VMEM((3, CHUNK, DH), jnp.float32),
            pltpu.VMEM((16, DH), jnp.float32),
            pltpu.VMEM_SHARED((SPROWS, DH), jnp.float32),
            pltpu.SemaphoreType.DMA,
            pltpu.SemaphoreType.DMA,
            pltpu.SemaphoreType.DMA,
        ],
    )
    def k(xr0_hbm, xr1_hbm, gidx_hbm, dst_hbm, out_hbm,
          idx2d, ldst2d, rows3, zero_v, agg_sh, sem0, sem1, sem2):
        cid = lax.axis_index("c")
        sid = lax.axis_index("s")
        tid = cid * 16 + sid

        pltpu.sync_copy(gidx_hbm.at[tid], idx2d)
        pltpu.sync_copy(dst_hbm.at[tid], ldst2d)

        nz = DH // 16

        def zfill(i, c):
            zero_v[i // nz, pl.ds((i % nz) * 16, 16)] = jnp.zeros(
                (16,), jnp.float32)
            return c
        lax.fori_loop(0, 16 * nz, zfill, 0)

        base = sid * STRIPE

        def run_phase(q, xr_hbm):
            def zcopy(i, c):
                pltpu.sync_copy(zero_v, agg_sh.at[pl.ds(base + i * 16, 16)])
                return c
            lax.fori_loop(0, STRIPE // 16, zcopy, 0)

            @pl.when(sid == 15)
            def _():
                pltpu.sync_copy(zero_v, agg_sh.at[pl.ds(16 * STRIPE, 16)])
                pltpu.sync_copy(zero_v.at[pl.ds(0, 8)], agg_sh.at[pl.ds(N, 8)])
            plsc.subcore_barrier()

            pltpu.async_copy(xr_hbm.at[idx2d.at[0]], rows3.at[0], sem0)
            pltpu.async_copy(xr_hbm.at[idx2d.at[1]], rows3.at[1], sem1)

            def chunk_body(ch, carry):
                nn = ch + 2
                for b, sem in ((0, sem0), (1, sem1), (2, sem2)):
                    @pl.when((nn < NCHUNKS) & (nn % 3 == b))
                    def _(b=b, sem=sem):
                        pltpu.async_copy(xr_hbm.at[idx2d.at[nn]],
                                         rows3.at[b], sem)
                for b, sem in ((0, sem0), (1, sem1), (2, sem2)):
                    @pl.when(ch % 3 == b)
                    def _(b=b, sem=sem):
                        pltpu.make_async_copy(xr_hbm.at[pl.ds(0, CHUNK)],
                                              rows3.at[b], sem).wait()
                        pltpu.sync_copy(rows3.at[b],
                                        agg_sh.at[ldst2d.at[ch]], add=True)
                return carry
            lax.fori_loop(0, NCHUNKS, chunk_body, 0)
            plsc.subcore_barrier()

            obase = (cid * NPHASE + q) * N
            pltpu.sync_copy(agg_sh.at[pl.ds(base, STRIPE)],
                            out_hbm.at[pl.ds(obase + base, STRIPE)])

            @pl.when(sid == 15)
            def _():
                pltpu.sync_copy(
                    agg_sh.at[pl.ds(16 * STRIPE, N - 16 * STRIPE)],
                    out_hbm.at[pl.ds(obase + 16 * STRIPE, N - 16 * STRIPE)])
            plsc.subcore_barrier()

        run_phase(0, xr0_hbm)
        run_phase(1, xr1_hbm)

    return k(xr0, xr1, gidx_t, dst_t)


def _post_body(p_ref, x_ref, gid_ref,
               b_rel, res_W, res_b, bn_g, bn_b,
               att_w_row, att_b,
               fc1_W, fc1_b, bn1_g, bn1_b,
               fc2_W, fc2_b, bn2_g, bn2_b,
               fc3_W, fc3_b, bn3_g, bn3_b,
               out_W, out_b, o_ref):
    x = x_ref[...]
    p = p_ref[...]
    agg = jnp.concatenate(
        [p[0, q] + p[1, q] for q in range(NPHASE)], axis=1)
    h = jnp.maximum(agg + b_rel[...], 0.0)
    res = jnp.maximum(
        jnp.dot(x, res_W[...], preferred_element_type=jnp.float32)
        + res_b[...], 0.0)
    h = h + res
    m = jnp.mean(h, axis=0, keepdims=True)
    v = jnp.mean((h - m) * (h - m), axis=0, keepdims=True)
    h = (h - m) / jnp.sqrt(v + EPS) * bn_g[...] + bn_b[...]
    z = jnp.sum(h * att_w_row[...], axis=1, keepdims=True) + att_b[...]
    w = 1.0 / (1.0 + jnp.exp(-z))
    hw = h * w
    sel = (lax.broadcasted_iota(jnp.int32, (B, N), 0)
           == gid_ref[...]).astype(jnp.float32)
    g = jnp.dot(sel, hw, preferred_element_type=jnp.float32)

    def fc(t, Wk, bk, gk, btk):
        y = jnp.maximum(
            jnp.dot(t, Wk[...], preferred_element_type=jnp.float32)
            + bk[...], 0.0)
        mm = jnp.mean(y, axis=0, keepdims=True)
        vv = jnp.mean((y - mm) * (y - mm), axis=0, keepdims=True)
        return (y - mm) / jnp.sqrt(vv + EPS) * gk[...] + btk[...]

    h1 = fc(g, fc1_W, fc1_b, bn1_g, bn1_b)
    h2 = fc(h1, fc2_W, fc2_b, bn2_g, bn2_b)
    h3 = fc(h2, fc3_W, fc3_b, bn3_g, bn3_b)
    o_ref[...] = (jnp.dot(h3, out_W[...], preferred_element_type=jnp.float32)
                  + out_b[...])


def _postprocess(partials, x, gid2d, p):
    args = (
        partials, x, gid2d,
        p['b_rel'].reshape(1, D), p['res_W'], p['res_b'].reshape(1, D),
        p['bn_g'].reshape(1, D), p['bn_b'].reshape(1, D),
        p['att_W'].reshape(1, D), p['att_b'].reshape(1, 1),
        p['fc1_W'], p['fc1_b'].reshape(1, H),
        p['bn1_g'].reshape(1, H), p['bn1_b'].reshape(1, H),
        p['fc2_W'], p['fc2_b'].reshape(1, H),
        p['bn2_g'].reshape(1, H), p['bn2_b'].reshape(1, H),
        p['fc3_W'], p['fc3_b'].reshape(1, H),
        p['bn3_g'].reshape(1, H), p['bn3_b'].reshape(1, H),
        p['out_W'], p['out_b'].reshape(1, 1),
    )
    return pl.pallas_call(
        _post_body,
        out_shape=jax.ShapeDtypeStruct((B, 1), jnp.float32),
    )(*args)


def kernel(node_feats, params, edge_index, etype, graph_ids):
    gidx_t, dst_t = _edge_tables(edge_index[0], etype, edge_index[1])
    xr0, xr1 = _rel_transform(node_feats, params['W_rel'])
    partials = _sc_aggregate(xr0, xr1, gidx_t, dst_t)
    gid2d = graph_ids.reshape(1, N)
    return _postprocess(partials.reshape(2, NPHASE, N, DH),
                        node_feats, gid2d, params)
